# baseline (speedup 1.0000x reference)
import dataclasses
import functools

import jax
import jax.numpy as jnp
from jax import lax
from jax.experimental import pallas as pl
from jax.experimental.pallas import tpu as pltpu
from jax.experimental.pallas import tpu_sc as plsc

_TOP_K = 8
_LANES = 16


def _matmul_body(x_ref, w_ref, b_ref, g_ref):
    g_ref[...] = (
        lax.dot_general(
            x_ref[...],
            w_ref[...],
            dimension_numbers=(((1,), (1,)), ((), ())),
            preferred_element_type=jnp.float32,
        )
        + b_ref[...]
    )


@functools.partial(jax.jit, static_argnames=("block_t", "off", "n_rows"))
def _gate_matmul(x, W, b, block_t=4096, off=0, n_rows=None):
    n_tokens, hidden = x.shape
    n_experts = W.shape[0]
    if n_rows is None:
        n_rows = n_tokens
    off_blocks = off // block_t
    return pl.pallas_call(
        _matmul_body,
        grid=(n_rows // block_t,),
        in_specs=[
            pl.BlockSpec((block_t, hidden), lambda i: (i + off_blocks, 0)),
            pl.BlockSpec((n_experts, hidden), lambda i: (0, 0)),
            pl.BlockSpec((1, n_experts), lambda i: (0, 0)),
        ],
        out_specs=pl.BlockSpec((block_t, n_experts), lambda i: (i, 0)),
        out_shape=jax.ShapeDtypeStruct((n_rows, n_experts), jnp.float32),
    )(x, W, b.reshape(1, n_experts))


def _sc_compiler_params():
    cp = pltpu.CompilerParams()
    if "needs_layout_passes" in pltpu.CompilerParams.__dataclass_fields__:
        cp = dataclasses.replace(cp, needs_layout_passes=False)
    return cp


def _merge_desc(av, ai, bv, bi):
    brv = lax.rev(bv, (0,))
    bri = lax.rev(bi, (0,))
    c = av >= brv
    lv = jnp.where(c, av, brv)
    li = jnp.where(c, ai, bri)
    return plsc.sort_key_val(lv, li, descending=True)


@functools.partial(jax.jit, static_argnames=("chunk",))
def _sc_topk(g, chunk=256):
    n_tokens, n_experts = g.shape
    mesh = plsc.VectorSubcoreMesh(core_axis_name="c", subcore_axis_name="s")

    @functools.partial(
        pl.kernel,
        out_type=[
            jax.ShapeDtypeStruct((n_tokens // 2, _LANES), jnp.float32),
            jax.ShapeDtypeStruct((n_tokens // 2, _LANES), jnp.int32),
        ],
        mesh=mesh,
        compiler_params=_sc_compiler_params(),
    )
    def sc_kernel(g_hbm, w_hbm, i_hbm):
        lane = lax.iota(jnp.int32, _LANES)
        low8 = lane < _TOP_K
        zero16 = jnp.zeros((_LANES,), jnp.float32)
        swap8 = lane ^ 8

        def top8(g_vmem, t):
            v0 = g_vmem[t, pl.ds(0, _LANES)]
            v1 = g_vmem[t, pl.ds(_LANES, _LANES)]
            v2 = g_vmem[t, pl.ds(2 * _LANES, _LANES)]
            v3 = g_vmem[t, pl.ds(3 * _LANES, _LANES)]
            s0v, s0i = plsc.sort_key_val(v0, lane, descending=True)
            s1v, s1i = plsc.sort_key_val(v1, lane + _LANES, descending=True)
            s2v, s2i = plsc.sort_key_val(v2, lane + 2 * _LANES, descending=True)
            s3v, s3i = plsc.sort_key_val(v3, lane + 3 * _LANES, descending=True)
            m01v, m01i = _merge_desc(s0v, s0i, s1v, s1i)
            m23v, m23i = _merge_desc(s2v, s2i, s3v, s3i)
            fv, fi = _merge_desc(m01v, m01i, m23v, m23i)
            top = plsc.cummax(fv)
            e = jnp.exp(fv - top)
            em = jnp.where(low8, e, zero16)
            total = jnp.sum(em, axis=0)
            return em / total, fi

        def body(g_vmem, w_vmem, i_vmem):
            @pl.loop(0, chunk // 4)
            def _(q):
                for u in range(2):
                    p = 2 * q + u
                    wa, ia = top8(g_vmem, 2 * p)
                    wb, ib = top8(g_vmem, 2 * p + 1)
                    _, wbs = plsc.sort_key_val(swap8, wb)
                    _, ibs = plsc.sort_key_val(swap8, ib)
                    w_vmem[p, :] = jnp.where(low8, wa, wbs)
                    i_vmem[p, :] = jnp.where(low8, ia, ibs)

        pltpu.emit_pipeline(
            body,
            grid=(n_tokens // chunk,),
            in_specs=[
                pl.BlockSpec((chunk, n_experts), lambda i: (i, 0)),
            ],
            out_specs=[
                pl.BlockSpec((chunk // 2, _LANES), lambda i: (i, 0)),
                pl.BlockSpec((chunk // 2, _LANES), lambda i: (i, 0)),
            ],
            core_axis_name=("c", "s"),
            dimension_semantics=(pltpu.PARALLEL,),
        )(g_hbm, w_hbm, i_hbm)

    return sc_kernel(g)


def kernel(x, W, b, n_chunks=4):
    n_tokens = x.shape[0]
    step = n_tokens // n_chunks
    ws, is_ = [], []
    for c in range(n_chunks):
        g = _gate_matmul(x, W, b, off=c * step, n_rows=step)
        w2, i2 = _sc_topk(g)
        ws.append(w2.reshape(step, _TOP_K))
        is_.append(i2.reshape(step, _TOP_K))
    return jnp.concatenate(ws, axis=0), jnp.concatenate(is_, axis=0)


# speedup vs baseline: 3.0718x; 3.0718x over previous
import functools

import jax
import jax.numpy as jnp
from jax.experimental import pallas as pl
from jax.experimental.pallas import tpu as pltpu

_TOP_K = 8


def _gate_topk_body(x_ref, w_ref, b_ref, w_out_ref, i_out_ref):
    g = (
        jax.lax.dot_general(
            w_ref[...],
            x_ref[...],
            dimension_numbers=(((1,), (1,)), ((), ())),
            preferred_element_type=jnp.float32,
        )
        + b_ref[...]
    )
    sub = jax.lax.broadcasted_iota(jnp.int32, g.shape, 0)
    cur = g
    vals = []
    idxs = []
    for k in range(_TOP_K):
        v, i = cur, sub
        while v.shape[0] > 1:
            h = v.shape[0] // 2
            c = v[:h] >= v[h:]
            v = jnp.where(c, v[:h], v[h:])
            i = jnp.where(c, i[:h], i[h:])
        vals.append(v)
        idxs.append(i)
        if k + 1 < _TOP_K:
            cur = jnp.where(sub == i, -jnp.inf, cur)
    v = jnp.concatenate(vals, axis=0)
    ew = jnp.exp(v - v[0:1, :])
    w_out_ref[...] = ew / jnp.sum(ew, axis=0, keepdims=True)
    i_out_ref[...] = jnp.concatenate(idxs, axis=0)


@functools.partial(jax.jit, static_argnames=("block_t", "interpret"))
def _gate_topk(x, W, b, block_t=4096, interpret=False):
    n_tokens, hidden = x.shape
    n_experts = W.shape[0]
    b2 = b.reshape(n_experts, 1)
    grid = (n_tokens // block_t,)
    w_out, i_out = pl.pallas_call(
        _gate_topk_body,
        grid=grid,
        in_specs=[
            pl.BlockSpec((block_t, hidden), lambda i: (i, 0)),
            pl.BlockSpec((n_experts, hidden), lambda i: (0, 0)),
            pl.BlockSpec((n_experts, 1), lambda i: (0, 0)),
        ],
        out_specs=[
            pl.BlockSpec((_TOP_K, block_t), lambda i: (0, i)),
            pl.BlockSpec((_TOP_K, block_t), lambda i: (0, i)),
        ],
        out_shape=[
            jax.ShapeDtypeStruct((_TOP_K, n_tokens), jnp.float32),
            jax.ShapeDtypeStruct((_TOP_K, n_tokens), jnp.int32),
        ],
        interpret=interpret,
    )(x, W, b2)
    return w_out.T, i_out.T


def kernel(x, W, b):
    return _gate_topk(x, W, b)
